# baseline (speedup 1.0000x reference)
import functools

import jax
import jax.numpy as jnp
from jax import lax
from jax.experimental import pallas as pl
from jax.experimental.pallas import tpu as pltpu
from jax.experimental.pallas import tpu_sc as plsc

VOCAB = 1000000
SEQ = 200
EMBED = 64
BATCH = 4096

NC = 2
NS = 16
NW = NC * NS
ROWS_PER_W = BATCH * SEQ // NW
LANES = 16
VPR = EMBED // LANES

BLK = 128
HBLK = BLK // 2
NBLK = ROWS_PER_W // BLK
NBUF = 4
AHEAD = 2
IAHEAD = 4
POS_REP = 320
VG = BLK // LANES

_mesh = plsc.VectorSubcoreMesh(core_axis_name="c", subcore_axis_name="s")


@functools.partial(
    pl.kernel,
    out_type=jax.ShapeDtypeStruct((BATCH * SEQ // 2, 2 * EMBED),
                                  jnp.float32),
    mesh=_mesh,
    compiler_params=pltpu.CompilerParams(needs_layout_passes=False),
    scratch_types=[
        pltpu.VMEM((POS_REP, EMBED), jnp.float32),
        [pltpu.VMEM((BLK, 2 * EMBED), jnp.float32) for _ in range(NBUF)],
        [pltpu.VMEM((BLK,), jnp.int32) for _ in range(NBUF)],
        [pltpu.VMEM((BLK,), jnp.int32) for _ in range(NBUF)],
        [pltpu.VMEM((BLK,), jnp.int32) for _ in range(NBUF)],
        [pltpu.SemaphoreType.DMA for _ in range(NBUF)],
        [pltpu.SemaphoreType.DMA for _ in range(NBUF)],
        [pltpu.SemaphoreType.DMA for _ in range(NBUF)],
    ],
)
def _seq_embed(seq_hbm, tok2_hbm, pos_hbm, out_hbm, pos_v, bufs, raws,
               pids, offs, isems, gsems, ssems):
    wid = lax.axis_index("s") * NC + lax.axis_index("c")
    base = wid * ROWS_PER_W
    pbase = base // 2

    pltpu.sync_copy(pos_hbm, pos_v.at[pl.ds(0, SEQ)])
    pltpu.sync_copy(pos_hbm.at[pl.ds(0, POS_REP - SEQ)],
                    pos_v.at[pl.ds(SEQ, POS_REP - SEQ)])

    def start_idx(b, slot):
        pltpu.async_copy(
            seq_hbm.at[pl.ds(pl.multiple_of(base + b * BLK, BLK), BLK)],
            raws[slot], isems[slot])

    def wait_idx(slot):
        pltpu.make_async_copy(
            seq_hbm.at[pl.ds(0, BLK)], raws[slot], isems[slot]).wait()

    def prep(slot):
        for g in range(VG):
            t = raws[slot][pl.ds(g * LANES, LANES)]
            pids[slot][pl.ds(g * LANES, LANES)] = lax.shift_right_logical(t, 1)
            offs[slot][pl.ds(g * LANES, LANES)] = lax.shift_left(
                lax.bitwise_and(t, 1), 6)

    def start_gather(slot):
        pltpu.async_copy(tok2_hbm.at[pids[slot]], bufs[slot], gsems[slot])

    def wait_gather(slot):
        pltpu.make_async_copy(
            tok2_hbm.at[pids[slot]], bufs[slot], gsems[slot]).wait()

    def start_scatter(b, slot):
        pltpu.async_copy(
            bufs[slot].at[pl.ds(0, HBLK)],
            out_hbm.at[pl.ds(pl.multiple_of(pbase + b * HBLK, HBLK), HBLK)],
            ssems[slot])

    def wait_scatter(slot):
        pltpu.make_async_copy(
            bufs[slot].at[pl.ds(0, HBLK)],
            out_hbm.at[pl.ds(0, HBLK)], ssems[slot]).wait()

    iota = lax.iota(jnp.int32, LANES)

    for s in range(IAHEAD):
        start_idx(s, s)
    for s in range(AHEAD):
        wait_idx(s)
        prep(s)
        start_gather(s)

    def group(g, carry):
        for s in range(NBUF):
            b = g * NBUF + s
            wait_gather(s)

            p0 = lax.rem(b * BLK, SEQ)
            buf = bufs[s]
            off_v = offs[s]

            def pair_rows(i, carry2):
                for par in range(2):
                    j = 2 * i + par
                    jv = jnp.full((LANES,), j, jnp.int32)
                    hv = plsc.load_gather(off_v, [jv])
                    for k in range(VPR):
                        pair = plsc.load_gather(
                            buf, [jv, hv + (iota + k * LANES)])
                        buf[i, pl.ds(par * EMBED + k * LANES, LANES)] = (
                            pair + pos_v[p0 + j, pl.ds(k * LANES, LANES)])
                return carry2

            lax.fori_loop(0, HBLK, pair_rows, 0, unroll=2)

            start_scatter(b, s)

            nb = b + AHEAD
            t = (s + AHEAD) % NBUF

            @pl.when(nb < NBLK)
            def _():
                @pl.when(nb >= NBUF)
                def _():
                    wait_scatter(t)
                wait_idx(t)
                prep(t)
                start_gather(t)

            fb = b + IAHEAD

            @pl.when(fb < NBLK)
            def _():
                start_idx(fb, s)

        return carry

    lax.fori_loop(0, NBLK // NBUF, group, 0)

    for b in range(NBLK - NBUF, NBLK):
        wait_scatter(b % NBUF)


def kernel(sequence, token_table, pos_table):
    seq_flat = sequence.reshape(-1).astype(jnp.int32)
    tok2 = token_table.reshape(VOCAB // 2, 2 * EMBED)
    out = _seq_embed(seq_flat, tok2, pos_table)
    return out.reshape(BATCH, SEQ, EMBED)


# speedup vs baseline: 1.0370x; 1.0370x over previous
import functools

import jax
import jax.numpy as jnp
from jax import lax
from jax.experimental import pallas as pl
from jax.experimental.pallas import tpu as pltpu
from jax.experimental.pallas import tpu_sc as plsc

VOCAB = 1000000
SEQ = 200
EMBED = 64
BATCH = 4096

NC = 2
NS = 16
NW = NC * NS
ROWS_PER_W = BATCH * SEQ // NW
LANES = 16
VPR = EMBED // LANES

BLK = 256
PBLK = BLK // 2
NBLK = ROWS_PER_W // BLK
POS_REP = 448

_mesh = plsc.VectorSubcoreMesh(core_axis_name="c", subcore_axis_name="s")


_WIDEN_BLK = 8000


def _widen_body(x_ref, o_ref):
    x = x_ref[...]
    o_ref[:, :EMBED] = x
    o_ref[:, EMBED:] = x


_widen = pl.pallas_call(
    _widen_body,
    grid=(VOCAB // _WIDEN_BLK,),
    in_specs=[pl.BlockSpec((_WIDEN_BLK, EMBED), lambda i: (i, 0))],
    out_specs=pl.BlockSpec((_WIDEN_BLK, 2 * EMBED), lambda i: (i, 0)),
    out_shape=jax.ShapeDtypeStruct((VOCAB, 2 * EMBED), jnp.float32),
)


@functools.partial(
    pl.kernel,
    out_type=jax.ShapeDtypeStruct((BATCH * SEQ // 2, 2 * EMBED),
                                  jnp.float32),
    mesh=_mesh,
    compiler_params=pltpu.CompilerParams(needs_layout_passes=False),
    scratch_types=[
        pltpu.VMEM((POS_REP, EMBED), jnp.float32),
        [pltpu.VMEM((BLK, 2 * EMBED), jnp.float32) for _ in range(2)],
        [pltpu.VMEM((BLK,), jnp.int32) for _ in range(2)],
        [pltpu.SemaphoreType.DMA for _ in range(2)],
        [pltpu.SemaphoreType.DMA for _ in range(2)],
        [pltpu.SemaphoreType.DMA for _ in range(2)],
    ],
)
def _gather_add(seq_hbm, tokw_hbm, pos_hbm, out_hbm, pos_v, bufs, idxs,
                isems, gsems, ssems):
    wid = lax.axis_index("s") * NC + lax.axis_index("c")
    base = wid * ROWS_PER_W
    pbase = base // 2

    for r0 in range(0, POS_REP, SEQ):
        n = min(SEQ, POS_REP - r0)
        pltpu.sync_copy(pos_hbm.at[pl.ds(0, n)], pos_v.at[pl.ds(r0, n)])

    SUBS = ((0, 128), (128, 128))

    def start_idx(b, slot):
        pltpu.async_copy(
            seq_hbm.at[pl.ds(pl.multiple_of(base + b * BLK, BLK), BLK)],
            idxs[slot], isems[slot])

    def wait_idx(slot):
        pltpu.make_async_copy(
            seq_hbm.at[pl.ds(0, BLK)], idxs[slot], isems[slot]).wait()

    def start_gather(slot):
        for o, n in SUBS:
            pltpu.async_copy(
                tokw_hbm.at[idxs[slot].at[pl.ds(o, n)]],
                bufs[slot].at[pl.ds(o, n)], gsems[slot])

    def wait_gather(slot):
        for o, n in SUBS:
            pltpu.make_async_copy(
                tokw_hbm.at[idxs[slot].at[pl.ds(o, n)]],
                bufs[slot].at[pl.ds(o, n)], gsems[slot]).wait()

    def start_scatter(b, slot):
        pltpu.async_copy(
            bufs[slot].at[pl.ds(0, PBLK)],
            out_hbm.at[pl.ds(pl.multiple_of(pbase + b * PBLK, PBLK), PBLK)],
            ssems[slot])

    def wait_scatter(slot):
        pltpu.make_async_copy(
            bufs[slot].at[pl.ds(0, PBLK)],
            out_hbm.at[pl.ds(0, PBLK)], ssems[slot]).wait()

    start_idx(0, 0)
    wait_idx(0)
    start_gather(0)
    start_idx(1, 1)

    def step(b, s):
        o = 1 - s
        wait_gather(s)

        @pl.when(b + 2 < NBLK)
        def _():
            start_idx(b + 2, s)

        @pl.when(b + 1 < NBLK)
        def _():
            wait_idx(o)

            @pl.when(b >= 1)
            def _():
                wait_scatter(o)
            start_gather(o)

        p0 = lax.rem(b * BLK, SEQ)
        buf = bufs[s]

        def pair_rows(i, carry):
            for k in range(VPR):
                plsc.addupdate(
                    buf.at[i, pl.ds(k * LANES, LANES)],
                    pos_v[p0 + i, pl.ds(k * LANES, LANES)])
            for k in range(VPR):
                buf[i, pl.ds(EMBED + k * LANES, LANES)] = (
                    buf[i + PBLK, pl.ds(k * LANES, LANES)]
                    + pos_v[p0 + PBLK + i, pl.ds(k * LANES, LANES)])
            return carry

        lax.fori_loop(0, PBLK, pair_rows, 0, unroll=2)

        start_scatter(b, s)

    def group(g, carry):
        step(2 * g, 0)
        step(2 * g + 1, 1)
        return carry

    lax.fori_loop(0, NBLK // 2, group, 0)

    wait_scatter(0)
    wait_scatter(1)


_UNPACK_K = 8


def _unpack_body(x_ref, o_ref):
    for t in range(_UNPACK_K):
        x = x_ref[pl.ds(t * PBLK, PBLK), :]
        o_ref[pl.ds(t * BLK, PBLK), :] = x[:, :EMBED]
        o_ref[pl.ds(t * BLK + PBLK, PBLK), :] = x[:, EMBED:]


_unpack = pl.pallas_call(
    _unpack_body,
    grid=(BATCH * SEQ // 2 // (_UNPACK_K * PBLK),),
    in_specs=[pl.BlockSpec((_UNPACK_K * PBLK, 2 * EMBED), lambda i: (i, 0))],
    out_specs=pl.BlockSpec((_UNPACK_K * BLK, EMBED), lambda i: (i, 0)),
    out_shape=jax.ShapeDtypeStruct((BATCH * SEQ, EMBED), jnp.float32),
)


def kernel(sequence, token_table, pos_table):
    seq_flat = sequence.reshape(-1).astype(jnp.int32)
    tokw = _widen(token_table)
    packed = _gather_add(seq_flat, tokw, pos_table)
    out = _unpack(packed)
    return out.reshape(BATCH, SEQ, EMBED)


# speedup vs baseline: 1.0587x; 1.0209x over previous
import functools

import jax
import jax.numpy as jnp
from jax import lax
from jax.experimental import pallas as pl
from jax.experimental.pallas import tpu as pltpu
from jax.experimental.pallas import tpu_sc as plsc

VOCAB = 1000000
SEQ = 200
EMBED = 64
BATCH = 4096

NC = 2
NS = 16
NW = NC * NS
ROWS_PER_W = BATCH * SEQ // NW
LANES = 16
VPR = EMBED // LANES

BLK = 256
PBLK = BLK // 2
NBLK = ROWS_PER_W // BLK
POS_REP = 448

_mesh = plsc.VectorSubcoreMesh(core_axis_name="c", subcore_axis_name="s")


_WIDEN_BLK = 8000


def _widen_body(x_ref, o_ref):
    x = x_ref[...]
    o_ref[:, :EMBED] = x
    o_ref[:, EMBED:] = x


_widen = pl.pallas_call(
    _widen_body,
    grid=(VOCAB // _WIDEN_BLK,),
    in_specs=[pl.BlockSpec((_WIDEN_BLK, EMBED), lambda i: (i, 0))],
    out_specs=pl.BlockSpec((_WIDEN_BLK, 2 * EMBED), lambda i: (i, 0)),
    out_shape=jax.ShapeDtypeStruct((VOCAB, 2 * EMBED), jnp.float32),
)


@functools.partial(
    pl.kernel,
    out_type=jax.ShapeDtypeStruct((BATCH * SEQ // 2, 2 * EMBED),
                                  jnp.float32),
    mesh=_mesh,
    scratch_types=[
        pltpu.VMEM((POS_REP, EMBED), jnp.float32),
        [pltpu.VMEM((BLK, 2 * EMBED), jnp.float32) for _ in range(2)],
        [pltpu.VMEM((BLK,), jnp.int32) for _ in range(2)],
        [pltpu.SemaphoreType.DMA for _ in range(2)],
        [pltpu.SemaphoreType.DMA for _ in range(2)],
        [pltpu.SemaphoreType.DMA for _ in range(2)],
    ],
)
def _gather_add(seq_hbm, tokw_hbm, pos_hbm, out_hbm, pos_v, bufs, idxs,
                isems, gsems, ssems):
    wid = lax.axis_index("s") * NC + lax.axis_index("c")
    base = wid * ROWS_PER_W
    pbase = base // 2

    for r0 in range(0, POS_REP, SEQ):
        n = min(SEQ, POS_REP - r0)
        pltpu.sync_copy(pos_hbm.at[pl.ds(0, n)], pos_v.at[pl.ds(r0, n)])

    SUBS = ((0, 128), (128, 128))

    def start_idx(b, slot):
        pltpu.async_copy(
            seq_hbm.at[pl.ds(pl.multiple_of(base + b * BLK, BLK), BLK)],
            idxs[slot], isems[slot])

    def wait_idx(slot):
        pltpu.make_async_copy(
            seq_hbm.at[pl.ds(0, BLK)], idxs[slot], isems[slot]).wait()

    def start_gather(slot):
        for o, n in SUBS:
            pltpu.async_copy(
                tokw_hbm.at[idxs[slot].at[pl.ds(o, n)]],
                bufs[slot].at[pl.ds(o, n)], gsems[slot])

    def wait_gather(slot):
        for o, n in SUBS:
            pltpu.make_async_copy(
                tokw_hbm.at[idxs[slot].at[pl.ds(o, n)]],
                bufs[slot].at[pl.ds(o, n)], gsems[slot]).wait()

    def start_scatter(b, slot):
        pltpu.async_copy(
            bufs[slot].at[pl.ds(0, PBLK)],
            out_hbm.at[pl.ds(pl.multiple_of(pbase + b * PBLK, PBLK), PBLK)],
            ssems[slot])

    def wait_scatter(slot):
        pltpu.make_async_copy(
            bufs[slot].at[pl.ds(0, PBLK)],
            out_hbm.at[pl.ds(0, PBLK)], ssems[slot]).wait()

    start_idx(0, 0)
    wait_idx(0)
    start_gather(0)
    start_idx(1, 1)

    def step(b, s):
        o = 1 - s
        wait_gather(s)

        @pl.when(b + 2 < NBLK)
        def _():
            start_idx(b + 2, s)

        @pl.when(b + 1 < NBLK)
        def _():
            wait_idx(o)

            @pl.when(b >= 1)
            def _():
                wait_scatter(o)
            start_gather(o)

        p0 = lax.rem(b * BLK, SEQ)
        buf = bufs[s]

        def pair_rows(i, carry):
            for k in range(VPR):
                plsc.addupdate(
                    buf.at[i, pl.ds(k * LANES, LANES)],
                    pos_v[p0 + i, pl.ds(k * LANES, LANES)])
            for k in range(VPR):
                buf[i, pl.ds(EMBED + k * LANES, LANES)] = (
                    buf[i + PBLK, pl.ds(k * LANES, LANES)]
                    + pos_v[p0 + PBLK + i, pl.ds(k * LANES, LANES)])
            return carry

        lax.fori_loop(0, PBLK, pair_rows, 0, unroll=2)

        start_scatter(b, s)

    def group(g, carry):
        step(2 * g, 0)
        step(2 * g + 1, 1)
        return carry

    lax.fori_loop(0, NBLK // 2, group, 0)

    wait_scatter(0)
    wait_scatter(1)


_UNPACK_B = 32
_UNPACK_ROWS = _UNPACK_B * SEQ
_UNPACK_CH = _UNPACK_ROWS // BLK


def _unpack_body(x_ref, o_ref):
    for t in range(_UNPACK_CH):
        x = x_ref[t * PBLK:(t + 1) * PBLK, :]
        for half, r0 in ((0, t * BLK), (1, t * BLK + PBLK)):
            src = x[:, half * EMBED:(half + 1) * EMBED]
            off = 0
            while off < PBLK:
                r = r0 + off
                b, s = r // SEQ, r % SEQ
                n = min(SEQ - s, PBLK - off)
                o_ref[b, s:s + n, :] = src[off:off + n, :]
                off += n


_unpack = pl.pallas_call(
    _unpack_body,
    grid=(BATCH // _UNPACK_B,),
    in_specs=[pl.BlockSpec((_UNPACK_ROWS // 2, 2 * EMBED), lambda i: (i, 0))],
    out_specs=pl.BlockSpec((_UNPACK_B, SEQ, EMBED), lambda i: (i, 0, 0)),
    out_shape=jax.ShapeDtypeStruct((BATCH, SEQ, EMBED), jnp.float32),
)


def kernel(sequence, token_table, pos_table):
    seq_flat = sequence.reshape(-1).astype(jnp.int32)
    tokw = _widen(token_table)
    packed = _gather_add(seq_flat, tokw, pos_table)
    return _unpack(packed)
